# baseline (speedup 1.0000x reference)
import functools

import jax
import jax.numpy as jnp
from jax import lax
from jax.experimental import pallas as pl
from jax.experimental.pallas import tpu as pltpu
from jax.experimental.pallas import tpu_sc as plsc

N_NODES = 10000
N_EDGES = 320000
F = 128

NC = 2
NS = 16
NW = NC * NS
CHUNK = 32
CPW = 320
NQ = 4
QC = CPW // NQ
E_PAD = NW * CPW * CHUNK
ACC_ROWS = 10240
RPT_Z = ACC_ROWS // NS
RPT_O = 624
TAIL_O = N_NODES - NS * RPT_O
ROW_BLK = N_NODES // 10


def _matmul_body(x_ref, w_ref, o_ref):
    o_ref[...] = jnp.dot(x_ref[...], w_ref[...],
                         preferred_element_type=jnp.float32)


def _matmul(x, w):
    return pl.pallas_call(
        _matmul_body,
        grid=(10,),
        in_specs=[
            pl.BlockSpec((ROW_BLK, F), lambda i: (i, 0)),
            pl.BlockSpec((F, F), lambda i: (0, 0)),
        ],
        out_specs=pl.BlockSpec((ROW_BLK, F), lambda i: (i, 0)),
        out_shape=jax.ShapeDtypeStruct((N_NODES, F), jnp.float32),
    )(x, w)


def _combine_body(p_ref, b_ref, o_ref):
    o_ref[...] = p_ref[0] + p_ref[1] + b_ref[...]


def _combine(p, b2d):
    return pl.pallas_call(
        _combine_body,
        grid=(10,),
        in_specs=[
            pl.BlockSpec((NC, ROW_BLK, F), lambda i: (0, i, 0)),
            pl.BlockSpec((1, F), lambda i: (0, 0)),
        ],
        out_specs=pl.BlockSpec((ROW_BLK, F), lambda i: (i, 0)),
        out_shape=jax.ShapeDtypeStruct((N_NODES, F), jnp.float32),
    )(p, b2d)


NBUF = 2
NGRP_Q = QC // NBUF


def _sc_scatter(h, sd2d, zeros):
    mesh = plsc.VectorSubcoreMesh(core_axis_name="c", subcore_axis_name="s")

    @functools.partial(
        pl.kernel,
        mesh=mesh,
        out_type=jax.ShapeDtypeStruct((NC, N_NODES, F), jnp.float32),
        scratch_types=[
            pltpu.VMEM((QC, 2, CHUNK), jnp.int32),
            pltpu.VMEM((NBUF, CHUNK, F), jnp.float32),
            pltpu.VMEM_SHARED((ACC_ROWS, F), jnp.float32),
            [pltpu.SemaphoreType.DMA] * NBUF,
        ],
    )
    def k(h_hbm, sd_hbm, z_hbm, out_hbm, idx_v, rows_v, acc, gsem):
        cid = lax.axis_index("c")
        sid = lax.axis_index("s")
        wid = sid * NC + cid

        pltpu.sync_copy(z_hbm.at[pl.ds(sid * RPT_Z, RPT_Z)],
                        acc.at[pl.ds(sid * RPT_Z, RPT_Z)])
        plsc.subcore_barrier()

        def _gather(c, b):
            return pltpu.make_async_copy(h_hbm.at[idx_v.at[c, 0]],
                                         rows_v.at[b], gsem[b])

        for q in range(NQ):
            pltpu.sync_copy(sd_hbm.at[pl.ds(wid * CPW + q * QC, QC)],
                            idx_v)

            for b in range(NBUF):
                _gather(b, b).start()

            def body(g, carry):
                c0 = g * NBUF
                for b in range(NBUF):
                    _gather(c0 + b, b).wait()
                    pltpu.sync_copy(rows_v.at[b],
                                    acc.at[idx_v.at[c0 + b, 1]], add=True)

                    @pl.when(g + 1 < NGRP_Q)
                    def _():
                        _gather(c0 + NBUF + b, b).start()

                return carry

            lax.fori_loop(0, NGRP_Q, body, None)

        plsc.subcore_barrier()

        pltpu.sync_copy(acc.at[pl.ds(sid * RPT_O, RPT_O)],
                        out_hbm.at[cid, pl.ds(sid * RPT_O, RPT_O)])

        @pl.when(sid == 0)
        def _():
            pltpu.sync_copy(acc.at[pl.ds(NS * RPT_O, TAIL_O)],
                            out_hbm.at[cid, pl.ds(NS * RPT_O, TAIL_O)])

    return k(h, sd2d, zeros)


def kernel(edge_index, features, weight, bias):
    ei = edge_index.astype(jnp.int32)
    pad = E_PAD - N_EDGES
    src = jnp.concatenate([ei[0], jnp.arange(pad, dtype=jnp.int32) % N_NODES])
    dst = jnp.concatenate([ei[1], jnp.full((pad,), N_NODES, jnp.int32)])
    sd2d = jnp.stack([src.reshape(NW * CPW, CHUNK),
                      dst.reshape(NW * CPW, CHUNK)], axis=1)
    h = _matmul(features, weight)
    zeros = jnp.zeros((ACC_ROWS, F), jnp.float32)
    p = _sc_scatter(h, sd2d, zeros)
    return _combine(p, bias.reshape(1, F))


# speedup vs baseline: 1.3426x; 1.3426x over previous
import functools

import jax
import jax.numpy as jnp
from jax import lax
from jax.experimental import pallas as pl
from jax.experimental.pallas import tpu as pltpu
from jax.experimental.pallas import tpu_sc as plsc

N_NODES = 10000
N_EDGES = 320000
F = 128

NC = 2
NS = 16
NW = NC * NS
CHUNK = 64
CPW = 160
NQ = 4
QC = CPW // NQ
E_PAD = NW * CPW * CHUNK
ACC_ROWS = 10240
RPT_Z = ACC_ROWS // NS
RPT_O = 624
TAIL_O = N_NODES - NS * RPT_O
ROW_BLK = N_NODES // 10


def _matmul_body(x_ref, w_ref, o_ref):
    o_ref[...] = jnp.dot(x_ref[...], w_ref[...],
                         preferred_element_type=jnp.float32)


def _matmul(x, w):
    return pl.pallas_call(
        _matmul_body,
        grid=(10,),
        in_specs=[
            pl.BlockSpec((ROW_BLK, F), lambda i: (i, 0)),
            pl.BlockSpec((F, F), lambda i: (0, 0)),
        ],
        out_specs=pl.BlockSpec((ROW_BLK, F), lambda i: (i, 0)),
        out_shape=jax.ShapeDtypeStruct((N_NODES, F), jnp.float32),
    )(x, w)


def _combine_body(p_ref, b_ref, o_ref):
    o_ref[...] = p_ref[0] + p_ref[1] + b_ref[...]


def _combine(p, b2d):
    return pl.pallas_call(
        _combine_body,
        grid=(10,),
        in_specs=[
            pl.BlockSpec((NC, ROW_BLK, F), lambda i: (0, i, 0)),
            pl.BlockSpec((1, F), lambda i: (0, 0)),
        ],
        out_specs=pl.BlockSpec((ROW_BLK, F), lambda i: (i, 0)),
        out_shape=jax.ShapeDtypeStruct((N_NODES, F), jnp.float32),
    )(p, b2d)


NBUF = 2
NGRP_Q = QC // NBUF


def _sc_scatter(h, sd2d, zeros):
    mesh = plsc.VectorSubcoreMesh(core_axis_name="c", subcore_axis_name="s")

    @functools.partial(
        pl.kernel,
        mesh=mesh,
        out_type=jax.ShapeDtypeStruct((NC, N_NODES, F), jnp.float32),
        scratch_types=[
            pltpu.VMEM((QC, 2, CHUNK), jnp.int32),
            pltpu.VMEM((NBUF, CHUNK, F), jnp.float32),
            pltpu.VMEM_SHARED((ACC_ROWS, F), jnp.float32),
            [pltpu.SemaphoreType.DMA] * NBUF,
        ],
    )
    def k(h_hbm, sd_hbm, z_hbm, out_hbm, idx_v, rows_v, acc, gsem):
        cid = lax.axis_index("c")
        sid = lax.axis_index("s")
        wid = sid * NC + cid

        pltpu.sync_copy(z_hbm.at[pl.ds(sid * RPT_Z, RPT_Z)],
                        acc.at[pl.ds(sid * RPT_Z, RPT_Z)])
        plsc.subcore_barrier()

        def _gather(c, b):
            return pltpu.make_async_copy(h_hbm.at[idx_v.at[c, 0]],
                                         rows_v.at[b], gsem[b])

        for q in range(NQ):
            pltpu.sync_copy(sd_hbm.at[pl.ds(wid * CPW + q * QC, QC)],
                            idx_v)

            for b in range(NBUF):
                _gather(b, b).start()

            def body(g, carry):
                c0 = g * NBUF
                for b in range(NBUF):
                    _gather(c0 + b, b).wait()
                    pltpu.sync_copy(rows_v.at[b],
                                    acc.at[idx_v.at[c0 + b, 1]], add=True)

                    @pl.when(g + 1 < NGRP_Q)
                    def _():
                        _gather(c0 + NBUF + b, b).start()

                return carry

            lax.fori_loop(0, NGRP_Q, body, None)

        plsc.subcore_barrier()

        pltpu.sync_copy(acc.at[pl.ds(sid * RPT_O, RPT_O)],
                        out_hbm.at[cid, pl.ds(sid * RPT_O, RPT_O)])

        @pl.when(sid == 0)
        def _():
            pltpu.sync_copy(acc.at[pl.ds(NS * RPT_O, TAIL_O)],
                            out_hbm.at[cid, pl.ds(NS * RPT_O, TAIL_O)])

    return k(h, sd2d, zeros)


def kernel(edge_index, features, weight, bias):
    ei = edge_index.astype(jnp.int32)
    pad = E_PAD - N_EDGES
    src = jnp.concatenate([ei[0], jnp.arange(pad, dtype=jnp.int32) % N_NODES])
    dst = jnp.concatenate([ei[1], jnp.full((pad,), N_NODES, jnp.int32)])
    sd2d = jnp.stack([src.reshape(NW * CPW, CHUNK),
                      dst.reshape(NW * CPW, CHUNK)], axis=1)
    h = _matmul(features, weight)
    zeros = jnp.zeros((ACC_ROWS, F), jnp.float32)
    p = _sc_scatter(h, sd2d, zeros)
    return _combine(p, bias.reshape(1, F))


# speedup vs baseline: 1.5143x; 1.1279x over previous
import functools

import jax
import jax.numpy as jnp
from jax import lax
from jax.experimental import pallas as pl
from jax.experimental.pallas import tpu as pltpu
from jax.experimental.pallas import tpu_sc as plsc

N_NODES = 10000
N_EDGES = 320000
F = 128

NC = 2
NS = 16
NW = NC * NS
CHUNK = 125
CPW = 80
NQ = 4
QC = CPW // NQ
ACC_ROWS = 10112
RPT_Z = ACC_ROWS // NS
RPT_O = 624
TAIL_O = N_NODES - NS * RPT_O
ROW_BLK = N_NODES // 10


def _matmul_body(x_ref, w_ref, o_ref):
    o_ref[...] = jnp.dot(x_ref[...], w_ref[...],
                         preferred_element_type=jnp.float32)


def _matmul(x, w):
    return pl.pallas_call(
        _matmul_body,
        grid=(10,),
        in_specs=[
            pl.BlockSpec((ROW_BLK, F), lambda i: (i, 0)),
            pl.BlockSpec((F, F), lambda i: (0, 0)),
        ],
        out_specs=pl.BlockSpec((ROW_BLK, F), lambda i: (i, 0)),
        out_shape=jax.ShapeDtypeStruct((N_NODES, F), jnp.float32),
    )(x, w)


def _combine_body(p_ref, b_ref, o_ref):
    o_ref[...] = p_ref[0] + p_ref[1] + b_ref[...]


def _combine(p, b2d):
    return pl.pallas_call(
        _combine_body,
        grid=(10,),
        in_specs=[
            pl.BlockSpec((NC, ROW_BLK, F), lambda i: (0, i, 0)),
            pl.BlockSpec((1, F), lambda i: (0, 0)),
        ],
        out_specs=pl.BlockSpec((ROW_BLK, F), lambda i: (i, 0)),
        out_shape=jax.ShapeDtypeStruct((N_NODES, F), jnp.float32),
    )(p, b2d)


NBUF = 2
NGRP_Q = QC // NBUF


def _sc_scatter(h, sd2d, zeros):
    mesh = plsc.VectorSubcoreMesh(core_axis_name="c", subcore_axis_name="s")

    @functools.partial(
        pl.kernel,
        mesh=mesh,
        out_type=jax.ShapeDtypeStruct((NC, N_NODES, F), jnp.float32),
        scratch_types=[
            pltpu.VMEM((QC, 2, CHUNK), jnp.int32),
            pltpu.VMEM((NBUF, CHUNK, F), jnp.float32),
            pltpu.VMEM_SHARED((ACC_ROWS, F), jnp.float32),
            [pltpu.SemaphoreType.DMA] * NBUF,
        ],
    )
    def k(h_hbm, sd_hbm, z_hbm, out_hbm, idx_v, rows_v, acc, gsem):
        cid = lax.axis_index("c")
        sid = lax.axis_index("s")
        wid = sid * NC + cid

        pltpu.sync_copy(z_hbm.at[pl.ds(sid * RPT_Z, RPT_Z)],
                        acc.at[pl.ds(sid * RPT_Z, RPT_Z)])
        plsc.subcore_barrier()

        def _gather(c, b):
            return pltpu.make_async_copy(h_hbm.at[idx_v.at[c, 0]],
                                         rows_v.at[b], gsem[b])

        for q in range(NQ):
            pltpu.sync_copy(sd_hbm.at[pl.ds(wid * CPW + q * QC, QC)],
                            idx_v)

            for b in range(NBUF):
                _gather(b, b).start()

            def body(g, carry):
                c0 = g * NBUF
                for b in range(NBUF):
                    _gather(c0 + b, b).wait()
                    pltpu.sync_copy(rows_v.at[b],
                                    acc.at[idx_v.at[c0 + b, 1]], add=True)

                    @pl.when(g + 1 < NGRP_Q)
                    def _():
                        _gather(c0 + NBUF + b, b).start()

                return carry

            lax.fori_loop(0, NGRP_Q, body, None)

        plsc.subcore_barrier()

        pltpu.sync_copy(acc.at[pl.ds(sid * RPT_O, RPT_O)],
                        out_hbm.at[cid, pl.ds(sid * RPT_O, RPT_O)])

        @pl.when(sid == 0)
        def _():
            pltpu.sync_copy(acc.at[pl.ds(NS * RPT_O, TAIL_O)],
                            out_hbm.at[cid, pl.ds(NS * RPT_O, TAIL_O)])

    return k(h, sd2d, zeros)


def kernel(edge_index, features, weight, bias):
    ei = edge_index.astype(jnp.int32)
    sd2d = jnp.stack([ei[0].reshape(NW * CPW, CHUNK),
                      ei[1].reshape(NW * CPW, CHUNK)], axis=1)
    h = _matmul(features, weight)
    zeros = jnp.zeros((ACC_ROWS, F), jnp.float32)
    p = _sc_scatter(h, sd2d, zeros)
    return _combine(p, bias.reshape(1, F))


# speedup vs baseline: 1.5166x; 1.0015x over previous
import functools

import jax
import jax.numpy as jnp
from jax import lax
from jax.experimental import pallas as pl
from jax.experimental.pallas import tpu as pltpu
from jax.experimental.pallas import tpu_sc as plsc

N_NODES = 10000
N_EDGES = 320000
F = 128

NC = 2
NS = 16
NW = NC * NS
CHUNK = 125
CPW = 80
NQ = 4
QC = CPW // NQ
ACC_ROWS = 10112
RPT_Z = ACC_ROWS // NS
RPT_O = 624
TAIL_O = N_NODES - NS * RPT_O
ROW_BLK = N_NODES // 10


def _matmul_body(x_ref, w_ref, o_ref):
    o_ref[...] = jnp.dot(x_ref[...], w_ref[...],
                         preferred_element_type=jnp.float32)


def _matmul(x, w):
    return pl.pallas_call(
        _matmul_body,
        grid=(10,),
        in_specs=[
            pl.BlockSpec((ROW_BLK, F), lambda i: (i, 0)),
            pl.BlockSpec((F, F), lambda i: (0, 0)),
        ],
        out_specs=pl.BlockSpec((ROW_BLK, F), lambda i: (i, 0)),
        out_shape=jax.ShapeDtypeStruct((N_NODES, F), jnp.float32),
    )(x, w)


def _combine_body(p_ref, b_ref, o_ref):
    o_ref[...] = p_ref[0] + p_ref[1] + b_ref[...]


def _combine(p, b2d):
    return pl.pallas_call(
        _combine_body,
        grid=(10,),
        in_specs=[
            pl.BlockSpec((NC, ROW_BLK, F), lambda i: (0, i, 0)),
            pl.BlockSpec((1, F), lambda i: (0, 0)),
        ],
        out_specs=pl.BlockSpec((ROW_BLK, F), lambda i: (i, 0)),
        out_shape=jax.ShapeDtypeStruct((N_NODES, F), jnp.float32),
    )(p, b2d)


NBUF = 2
NGRP_Q = QC // NBUF


def _sc_scatter(h, sd2d, zeros):
    mesh = plsc.VectorSubcoreMesh(core_axis_name="c", subcore_axis_name="s")

    @functools.partial(
        pl.kernel,
        mesh=mesh,
        out_type=jax.ShapeDtypeStruct((NC, N_NODES, F), jnp.float32),
        scratch_types=[
            pltpu.VMEM((QC, 2, CHUNK), jnp.int32),
            pltpu.VMEM((NBUF, CHUNK, F), jnp.float32),
            pltpu.VMEM_SHARED((ACC_ROWS, F), jnp.float32),
            [pltpu.SemaphoreType.DMA] * NBUF,
        ],
    )
    def k(h_hbm, sd_hbm, z_hbm, out_hbm, idx_v, rows_v, acc, gsem):
        cid = lax.axis_index("c")
        sid = lax.axis_index("s")
        wid = sid * NC + cid

        pltpu.sync_copy(z_hbm, acc.at[pl.ds(sid * RPT_Z, RPT_Z)])
        plsc.subcore_barrier()

        def _gather(c, b):
            return pltpu.make_async_copy(h_hbm.at[idx_v.at[c, 0]],
                                         rows_v.at[b], gsem[b])

        for q in range(NQ):
            pltpu.sync_copy(sd_hbm.at[pl.ds(wid * CPW + q * QC, QC)],
                            idx_v)

            for b in range(NBUF):
                _gather(b, b).start()

            def body(g, carry):
                c0 = g * NBUF
                for b in range(NBUF):
                    _gather(c0 + b, b).wait()
                    pltpu.sync_copy(rows_v.at[b],
                                    acc.at[idx_v.at[c0 + b, 1]], add=True)

                    @pl.when(g + 1 < NGRP_Q)
                    def _():
                        _gather(c0 + NBUF + b, b).start()

                return carry

            lax.fori_loop(0, NGRP_Q, body, None)

        plsc.subcore_barrier()

        pltpu.sync_copy(acc.at[pl.ds(sid * RPT_O, RPT_O)],
                        out_hbm.at[cid, pl.ds(sid * RPT_O, RPT_O)])

        @pl.when(sid == 0)
        def _():
            pltpu.sync_copy(acc.at[pl.ds(NS * RPT_O, TAIL_O)],
                            out_hbm.at[cid, pl.ds(NS * RPT_O, TAIL_O)])

    return k(h, sd2d, zeros)


def kernel(edge_index, features, weight, bias):
    ei = edge_index.astype(jnp.int32)
    sd2d = jnp.stack([ei[0].reshape(NW * CPW, CHUNK),
                      ei[1].reshape(NW * CPW, CHUNK)], axis=1)
    h = _matmul(features, weight)
    zeros = jnp.zeros((RPT_Z, F), jnp.float32)
    p = _sc_scatter(h, sd2d, zeros)
    return _combine(p, bias.reshape(1, F))


# speedup vs baseline: 1.5890x; 1.0478x over previous
import functools

import jax
import jax.numpy as jnp
from jax import lax
from jax.experimental import pallas as pl
from jax.experimental.pallas import tpu as pltpu
from jax.experimental.pallas import tpu_sc as plsc

N_NODES = 10000
N_EDGES = 320000
F = 128

NC = 2
NS = 16
NW = NC * NS
CHUNK = 96
CPW = 105
NQ = 5
QC = CPW // NQ
E_PAD = NW * CPW * CHUNK
ACC_ROWS = 10240
RPT_Z = ACC_ROWS // NS
RPT_O = 624
TAIL_O = N_NODES - NS * RPT_O
ROW_BLK = N_NODES // 10


def _matmul_body(x_ref, w_ref, o_ref):
    o_ref[...] = jnp.dot(x_ref[...], w_ref[...],
                         preferred_element_type=jnp.float32)


def _matmul(x, w):
    return pl.pallas_call(
        _matmul_body,
        grid=(10,),
        in_specs=[
            pl.BlockSpec((ROW_BLK, F), lambda i: (i, 0)),
            pl.BlockSpec((F, F), lambda i: (0, 0)),
        ],
        out_specs=pl.BlockSpec((ROW_BLK, F), lambda i: (i, 0)),
        out_shape=jax.ShapeDtypeStruct((N_NODES, F), jnp.float32),
    )(x, w)


def _combine_body(p_ref, b_ref, o_ref):
    o_ref[...] = p_ref[0] + p_ref[1] + b_ref[...]


def _combine(p, b2d):
    return pl.pallas_call(
        _combine_body,
        grid=(10,),
        in_specs=[
            pl.BlockSpec((NC, ROW_BLK, F), lambda i: (0, i, 0)),
            pl.BlockSpec((1, F), lambda i: (0, 0)),
        ],
        out_specs=pl.BlockSpec((ROW_BLK, F), lambda i: (i, 0)),
        out_shape=jax.ShapeDtypeStruct((N_NODES, F), jnp.float32),
    )(p, b2d)


NBUF = 3
NGRP_Q = QC // NBUF


def _sc_scatter(h, sd2d, zeros):
    mesh = plsc.VectorSubcoreMesh(core_axis_name="c", subcore_axis_name="s")

    @functools.partial(
        pl.kernel,
        mesh=mesh,
        out_type=jax.ShapeDtypeStruct((NC, N_NODES, F), jnp.float32),
        scratch_types=[
            pltpu.VMEM((QC, 2, CHUNK), jnp.int32),
            pltpu.VMEM((NBUF, CHUNK, F), jnp.float32),
            pltpu.VMEM_SHARED((ACC_ROWS, F), jnp.float32),
            [pltpu.SemaphoreType.DMA] * NBUF,
        ],
    )
    def k(h_hbm, sd_hbm, z_hbm, out_hbm, idx_v, rows_v, acc, gsem):
        cid = lax.axis_index("c")
        sid = lax.axis_index("s")
        wid = sid * NC + cid

        pltpu.sync_copy(z_hbm, acc.at[pl.ds(sid * RPT_Z, RPT_Z)])
        plsc.subcore_barrier()

        def _gather(c, b):
            return pltpu.make_async_copy(h_hbm.at[idx_v.at[c, 0]],
                                         rows_v.at[b], gsem[b])

        for q in range(NQ):
            pltpu.sync_copy(sd_hbm.at[pl.ds(wid * CPW + q * QC, QC)],
                            idx_v)

            for b in range(NBUF):
                _gather(b, b).start()

            def body(g, carry):
                c0 = g * NBUF
                for b in range(NBUF):
                    _gather(c0 + b, b).wait()
                    pltpu.sync_copy(rows_v.at[b],
                                    acc.at[idx_v.at[c0 + b, 1]], add=True)

                    @pl.when(g + 1 < NGRP_Q)
                    def _():
                        _gather(c0 + NBUF + b, b).start()

                return carry

            lax.fori_loop(0, NGRP_Q, body, None)

        plsc.subcore_barrier()

        pltpu.sync_copy(acc.at[pl.ds(sid * RPT_O, RPT_O)],
                        out_hbm.at[cid, pl.ds(sid * RPT_O, RPT_O)])

        @pl.when(sid == 0)
        def _():
            pltpu.sync_copy(acc.at[pl.ds(NS * RPT_O, TAIL_O)],
                            out_hbm.at[cid, pl.ds(NS * RPT_O, TAIL_O)])

    return k(h, sd2d, zeros)


def kernel(edge_index, features, weight, bias):
    ei = edge_index.astype(jnp.int32)
    pad = E_PAD - N_EDGES
    src = jnp.concatenate([ei[0], jnp.arange(pad, dtype=jnp.int32) % N_NODES])
    dst = jnp.concatenate([ei[1], jnp.full((pad,), N_NODES, jnp.int32)])
    sd2d = jnp.stack([src.reshape(NW * CPW, CHUNK),
                      dst.reshape(NW * CPW, CHUNK)], axis=1)
    h = _matmul(features, weight)
    zeros = jnp.zeros((RPT_Z, F), jnp.float32)
    p = _sc_scatter(h, sd2d, zeros)
    return _combine(p, bias.reshape(1, F))


# speedup vs baseline: 1.6921x; 1.0649x over previous
import functools

import jax
import jax.numpy as jnp
from jax import lax
from jax.experimental import pallas as pl
from jax.experimental.pallas import tpu as pltpu
from jax.experimental.pallas import tpu_sc as plsc

N_NODES = 10000
N_EDGES = 320000
F = 128

NC = 2
NS = 16
NW = NC * NS
CHUNK = 96
CPW = 105
NQ = 5
QC = CPW // NQ
E_PAD = NW * CPW * CHUNK
ACC_ROWS = 10112
RPT_Z = ACC_ROWS // NS
RPT_O = 624
TAIL_O = N_NODES - NS * RPT_O
ROW_BLK = N_NODES // 10


def _matmul_body(x_ref, w_ref, o_ref):
    o_ref[...] = jnp.dot(x_ref[...], w_ref[...],
                         preferred_element_type=jnp.float32)


def _matmul(x, w):
    return pl.pallas_call(
        _matmul_body,
        grid=(10,),
        in_specs=[
            pl.BlockSpec((ROW_BLK, F), lambda i: (i, 0)),
            pl.BlockSpec((F, F), lambda i: (0, 0)),
        ],
        out_specs=pl.BlockSpec((ROW_BLK, F), lambda i: (i, 0)),
        out_shape=jax.ShapeDtypeStruct((N_NODES, F), jnp.float32),
    )(x, w)


def _combine_body(p_ref, b_ref, o_ref):
    o_ref[...] = p_ref[0] + p_ref[1] + b_ref[...]


def _combine(p, b2d):
    return pl.pallas_call(
        _combine_body,
        grid=(10,),
        in_specs=[
            pl.BlockSpec((NC, ROW_BLK, F), lambda i: (0, i, 0)),
            pl.BlockSpec((1, F), lambda i: (0, 0)),
        ],
        out_specs=pl.BlockSpec((ROW_BLK, F), lambda i: (i, 0)),
        out_shape=jax.ShapeDtypeStruct((N_NODES, F), jnp.float32),
    )(p, b2d)


NBUF = 3
NGRP_Q = QC // NBUF


def _sc_scatter(h, sd2d, zeros):
    mesh = plsc.VectorSubcoreMesh(core_axis_name="c", subcore_axis_name="s")

    @functools.partial(
        pl.kernel,
        mesh=mesh,
        out_type=jax.ShapeDtypeStruct((NC, N_NODES, F), jnp.float32),
        scratch_types=[
            pltpu.VMEM((2, QC, 2, CHUNK), jnp.int32),
            pltpu.VMEM((NBUF, CHUNK, F), jnp.float32),
            pltpu.VMEM_SHARED((ACC_ROWS, F), jnp.float32),
            [pltpu.SemaphoreType.DMA] * NBUF,
            pltpu.SemaphoreType.DMA,
        ],
    )
    def k(h_hbm, sd_hbm, z_hbm, out_hbm, idx_v, rows_v, acc, gsem, isem):
        cid = lax.axis_index("c")
        sid = lax.axis_index("s")
        wid = sid * NC + cid

        pltpu.sync_copy(z_hbm, acc.at[pl.ds(sid * RPT_Z, RPT_Z)])
        plsc.subcore_barrier()

        def _gather(ip, c, b):
            return pltpu.make_async_copy(h_hbm.at[idx_v.at[ip, c, 0]],
                                         rows_v.at[b], gsem[b])

        def _idx_load(q, ip):
            return pltpu.make_async_copy(
                sd_hbm.at[pl.ds(wid * CPW + q * QC, QC)], idx_v.at[ip], isem)

        _idx_load(0, 0).start()
        _idx_load(0, 0).wait()
        for b in range(NBUF):
            _gather(0, b, b).start()

        for q in range(NQ):
            ip, ipn = q % 2, (q + 1) % 2
            if q + 1 < NQ:
                _idx_load(q + 1, ipn).start()

            def body(g, carry):
                c0 = g * NBUF
                for b in range(NBUF):
                    _gather(ip, c0 + b, b).wait()
                    pltpu.sync_copy(rows_v.at[b],
                                    acc.at[idx_v.at[ip, c0 + b, 1]], add=True)

                    if q + 1 < NQ:
                        @pl.when(g + 1 < NGRP_Q)
                        def _():
                            _gather(ip, c0 + NBUF + b, b).start()

                        @pl.when(g + 1 == NGRP_Q)
                        def _():
                            if b == 0:
                                _idx_load(q + 1, ipn).wait()
                            _gather(ipn, b, b).start()
                    else:
                        @pl.when(g + 1 < NGRP_Q)
                        def _():
                            _gather(ip, c0 + NBUF + b, b).start()

                return carry

            lax.fori_loop(0, NGRP_Q, body, None)

        plsc.subcore_barrier()

        pltpu.sync_copy(acc.at[pl.ds(sid * RPT_O, RPT_O)],
                        out_hbm.at[cid, pl.ds(sid * RPT_O, RPT_O)])

        @pl.when(sid == 0)
        def _():
            pltpu.sync_copy(acc.at[pl.ds(NS * RPT_O, TAIL_O)],
                            out_hbm.at[cid, pl.ds(NS * RPT_O, TAIL_O)])

    return k(h, sd2d, zeros)


def kernel(edge_index, features, weight, bias):
    ei = edge_index.astype(jnp.int32)
    pad = E_PAD - N_EDGES
    src = jnp.concatenate([ei[0], jnp.arange(pad, dtype=jnp.int32) % N_NODES])
    dst = jnp.concatenate([ei[1], jnp.full((pad,), N_NODES, jnp.int32)])
    sd2d = jnp.stack([src.reshape(NW * CPW, CHUNK),
                      dst.reshape(NW * CPW, CHUNK)], axis=1)
    h = _matmul(features, weight)
    zeros = jnp.zeros((RPT_Z, F), jnp.float32)
    p = _sc_scatter(h, sd2d, zeros)
    return _combine(p, bias.reshape(1, F))
